# baseline (speedup 1.0000x reference)
import functools
import jax
import jax.numpy as jnp
from jax import lax
from jax.experimental import pallas as pl
from jax.experimental.pallas import tpu as pltpu
from jax.experimental.pallas import tpu_sc as plsc

_W = 128
_NC = 2
_NS = 16


def _sc_body(embed_hbm, out_hbm, idx_v, rows_v, r_sh, sem, sem2):
    c = lax.axis_index("c")
    s = lax.axis_index("s")

    lane = lax.broadcasted_iota(jnp.int32, (16,), 0)
    for t in range(4):
        m = s * 64 + t * 16 + lane
        idx = jnp.clip(511 - m, -_W, _W) + _W
        idx_v[pl.ds(t * 16, 16)] = idx
    pltpu.async_copy(embed_hbm.at[idx_v], rows_v, sem).wait()
    pltpu.sync_copy(rows_v, r_sh.at[pl.ds(s * 64, 64)])
    plsc.subcore_barrier()

    w = s * _NC + c
    handles = []
    for k in range(32):
        p = w * 32 + k
        b = p // 512
        i = p % 512
        handles.append(
            pltpu.async_copy(r_sh.at[pl.ds(511 - i, 512)], out_hbm.at[b, i], sem2)
        )
    for h in handles:
        h.wait()


def kernel(x, embed):
    bsz, length, _ = x.shape
    d = embed.shape[1]
    mesh = plsc.VectorSubcoreMesh(core_axis_name="c", subcore_axis_name="s")
    run = functools.partial(
        pl.kernel,
        mesh=mesh,
        out_type=jax.ShapeDtypeStruct((bsz, length, length, d), jnp.float32),
        scratch_types=[
            pltpu.VMEM((64,), jnp.int32),
            pltpu.VMEM((64, d), jnp.float32),
            pltpu.VMEM_SHARED((1024, d), jnp.float32),
            pltpu.SemaphoreType.DMA,
            pltpu.SemaphoreType.DMA,
        ],
    )(_sc_body)
    return run(embed)
